# Initial kernel scaffold; baseline (speedup 1.0000x reference)
import jax
import jax.numpy as jnp
from jax.experimental import pallas as pl


def kernel(x, edge_index, enc_w0, enc_b0, enc_w1, enc_b1, dec_w0, dec_b0, dec_w1, dec_b1, dec_w2, dec_b2, mu_w, mu_b, var_w, var_b, ad_w, ad_b, d1_w, d1_b, d2_w, d2_b, ab_w, ab_b):
    raise NotImplementedError("write your pallas kernel here")


# speedup vs baseline: 2.0823x; 2.0823x over previous
import functools

import jax
import jax.numpy as jnp
from jax import lax
from jax.experimental import pallas as pl
from jax.experimental.pallas import tpu as pltpu
from jax.experimental.pallas import tpu_sc as plsc

N = 10000
E = 160000
D_IN = 128
D_OUT = 128
BIG = 256
H = 16
K = 6

F32 = jnp.float32


_BN_PRE = 1000


def _pre_enc_body(x_ref, w0t_ref, b0_ref, a_ref, b_ref):
    xb = x_ref[...]
    wa = w0t_ref[:D_IN, :] - w0t_ref[D_IN:, :]
    a_ref[...] = jnp.dot(xb, wa, preferred_element_type=F32) + b0_ref[...]
    b_ref[...] = jnp.dot(xb, w0t_ref[D_IN:, :], preferred_element_type=F32)


def _pre_enc(x, w0t, b0r):
    return pl.pallas_call(
        _pre_enc_body,
        grid=(N // _BN_PRE,),
        in_specs=[
            pl.BlockSpec((_BN_PRE, D_IN), lambda i: (i, 0)),
            pl.BlockSpec((2 * D_IN, BIG), lambda i: (0, 0)),
            pl.BlockSpec((1, BIG), lambda i: (0, 0)),
        ],
        out_specs=[
            pl.BlockSpec((_BN_PRE, BIG), lambda i: (i, 0)),
            pl.BlockSpec((_BN_PRE, BIG), lambda i: (i, 0)),
        ],
        out_shape=[
            jax.ShapeDtypeStruct((N, BIG), F32),
            jax.ShapeDtypeStruct((N, BIG), F32),
        ],
    )(x, w0t, b0r)


_BE_MM = 640


def _mm_relu_body(t_ref, wt_ref, b_ref, u_ref):
    acc = jnp.dot(t_ref[...], wt_ref[...], preferred_element_type=F32)
    u_ref[...] = jnp.maximum(acc + b_ref[...], 0.0)


def _mm_relu(t, wt, br):
    return pl.pallas_call(
        _mm_relu_body,
        grid=(E // _BE_MM,),
        in_specs=[
            pl.BlockSpec((_BE_MM, BIG), lambda i: (i, 0)),
            pl.BlockSpec((BIG, BIG), lambda i: (0, 0)),
            pl.BlockSpec((1, BIG), lambda i: (0, 0)),
        ],
        out_specs=pl.BlockSpec((_BE_MM, BIG), lambda i: (i, 0)),
        out_shape=jax.ShapeDtypeStruct((E, BIG), F32),
    )(t, wt, br)


_BN_POST = 400


def _post_enc_body(seg_ref, cnt_ref, cnt2_ref, eps_ref, adwt_ref, adb_ref, d1wt_ref, d1b_ref,
                   d2wt_ref, d2b_ref, abwt_ref, abb_ref, muwt_ref, mub_ref,
                   vwt_ref, vb_ref, d0t_ref, d0b_ref,
                   mu_ref, lv_ref, z0_ref, z_ref, ld_ref, a2_ref, b2_ref):
    cnt = cnt_ref[:, 0:1] + cnt2_ref[:, 0:1]
    inv = 1.0 / jnp.maximum(cnt, 1.0)
    h = seg_ref[...] * inv

    mu = jnp.dot(h, muwt_ref[...], preferred_element_type=F32) + mub_ref[...]
    lv = jnp.dot(h, vwt_ref[...], preferred_element_type=F32) + vb_ref[...]
    std = jnp.exp(0.5 * lv)
    z0 = mu + eps_ref[...] * std

    colc = lax.broadcasted_iota(jnp.int32, (H, BIG), 1)
    rowc = lax.broadcasted_iota(jnp.int32, (H, BIG), 0)
    Ri = (colc // H == rowc).astype(F32)
    Rj = (colc % H == rowc).astype(F32)
    coltc = lax.broadcasted_iota(jnp.int32, (BIG, H), 0)
    rowtc = lax.broadcasted_iota(jnp.int32, (BIG, H), 1)
    RiT = (coltc // H == rowtc).astype(F32)
    RjT = (coltc % H == rowtc).astype(F32)
    m1 = lax.broadcasted_iota(jnp.int32, (1, BIG), 1)
    Migtj = ((m1 // H) > (m1 % H)).astype(F32)
    Mjgti = ((m1 % H) > (m1 // H)).astype(F32)
    r16 = lax.broadcasted_iota(jnp.int32, (H, H), 0)
    c16 = lax.broadcasted_iota(jnp.int32, (H, H), 1)
    Pf = (r16 + c16 == H - 1).astype(F32)

    z = z0
    ld = jnp.zeros((_BN_POST, 1), F32)
    for k in range(K):
        F = jnp.dot(h, adwt_ref[k], preferred_element_type=F32) + adb_ref[k]
        d1 = jnp.tanh(jnp.dot(h, d1wt_ref[k], preferred_element_type=F32)
                      + d1b_ref[k])
        d2 = jnp.tanh(jnp.dot(h, d2wt_ref[k], preferred_element_type=F32)
                      + d2b_ref[k])
        bk = jnp.dot(h, abwt_ref[k], preferred_element_type=F32) + abb_ref[k]
        zp = jnp.dot(z, Pf, preferred_element_type=F32) if (k % 2 == 1) else z
        zrep = jnp.dot(zp, Ri, preferred_element_type=F32)
        r2qzb = (jnp.dot(F * Migtj * zrep, RjT, preferred_element_type=F32)
                 + zp * d2 + bk)
        th = jnp.tanh(r2qzb)
        trep = jnp.dot(th, Rj, preferred_element_type=F32)
        zmid = (jnp.dot(F * Mjgti * trep, RiT, preferred_element_type=F32)
                + th * d1)
        if k % 2 == 1:
            zmid = jnp.dot(zmid, Pf, preferred_element_type=F32)
        z = zmid + z
        dj = (1.0 - th * th) * d1 * d2 + 1.0
        ld = ld + jnp.sum(jnp.log(jnp.abs(dj)), axis=1, keepdims=True)

    d0t = d0t_ref[...]
    va = d0t[:H, :] - d0t[H:, :]
    a2 = jnp.dot(z, va, preferred_element_type=F32) + d0b_ref[...]
    b2 = jnp.dot(z, d0t[H:, :], preferred_element_type=F32)

    mu_ref[...] = mu
    lv_ref[...] = lv
    z0_ref[...] = z0
    z_ref[...] = z
    ld_ref[...] = ld * jnp.ones((1, 128), F32)
    a2_ref[...] = a2
    b2_ref[...] = b2


def _post_enc(seg, cnt, cnt2, eps, adwt, adb, d1wt, d1b, d2wt, d2b, abwt, abb,
              muwt, mub, vwt, vb, d0t, d0b):
    g = N // _BN_POST
    row = lambda w: pl.BlockSpec((_BN_POST, w), lambda i: (i, 0))
    full2 = lambda a, b: pl.BlockSpec((a, b), lambda i: (0, 0))
    full3 = lambda a, b, c: pl.BlockSpec((a, b, c), lambda i: (0, 0, 0))
    return pl.pallas_call(
        _post_enc_body,
        grid=(g,),
        in_specs=[
            row(BIG), row(128), row(128), row(H),
            full3(K, BIG, BIG), full3(K, 1, BIG),
            full3(K, BIG, H), full3(K, 1, H),
            full3(K, BIG, H), full3(K, 1, H),
            full3(K, BIG, H), full3(K, 1, H),
            full2(BIG, H), full2(1, H),
            full2(BIG, H), full2(1, H),
            full2(2 * H, BIG), full2(1, BIG),
        ],
        out_specs=[
            row(H), row(H), row(H), row(H), row(128), row(BIG), row(BIG),
        ],
        out_shape=[
            jax.ShapeDtypeStruct((N, H), F32),
            jax.ShapeDtypeStruct((N, H), F32),
            jax.ShapeDtypeStruct((N, H), F32),
            jax.ShapeDtypeStruct((N, H), F32),
            jax.ShapeDtypeStruct((N, 128), F32),
            jax.ShapeDtypeStruct((N, BIG), F32),
            jax.ShapeDtypeStruct((N, BIG), F32),
        ],
    )(seg, cnt, cnt2, eps, adwt, adb, d1wt, d1b, d2wt, d2b, abwt, abb,
      muwt, mub, vwt, vb, d0t, d0b)


def _dec_out_body(seg_ref, cnt_ref, cnt2_ref, wt_ref, b_ref, o_ref):
    cnt = cnt_ref[:, 0:1] + cnt2_ref[:, 0:1]
    inv = 1.0 / jnp.maximum(cnt, 1.0)
    gate = (cnt > 0.0).astype(F32)
    o_ref[...] = (jnp.dot(seg_ref[...] * inv, wt_ref[...],
                          preferred_element_type=F32)
                  + b_ref[...] * gate)


def _dec_out(seg, cnt, cnt2, wt, br):
    return pl.pallas_call(
        _dec_out_body,
        grid=(N // _BN_PRE,),
        in_specs=[
            pl.BlockSpec((_BN_PRE, BIG), lambda i: (i, 0)),
            pl.BlockSpec((_BN_PRE, 128), lambda i: (i, 0)),
            pl.BlockSpec((_BN_PRE, 128), lambda i: (i, 0)),
            pl.BlockSpec((BIG, D_OUT), lambda i: (0, 0)),
            pl.BlockSpec((1, D_OUT), lambda i: (0, 0)),
        ],
        out_specs=pl.BlockSpec((_BN_PRE, D_OUT), lambda i: (i, 0)),
        out_shape=jax.ShapeDtypeStruct((N, D_OUT), F32),
    )(seg, cnt, cnt2, wt, br)


_NWORK = 32
_EPW = E // _NWORK
_GCH = 128
_GFULL = _EPW // _GCH
_GTAIL = _EPW - _GFULL * _GCH

def _mesh():
    return plsc.VectorSubcoreMesh(core_axis_name="c", subcore_axis_name="s")


def _gather_body(a_hbm, b_hbm, dst_hbm, src_hbm, t_hbm, dvec, svec,
                 bufa, bufb):
    wid = lax.axis_index("s") * 2 + lax.axis_index("c")
    base = wid * _EPW
    pltpu.sync_copy(dst_hbm.at[pl.ds(base, _EPW)], dvec)
    pltpu.sync_copy(src_hbm.at[pl.ds(base, _EPW)], svec)

    def do_chunk(off, nrows):
        pltpu.sync_copy(a_hbm.at[dvec.at[pl.ds(off, nrows)]],
                        bufa.at[pl.ds(0, nrows)])
        pltpu.sync_copy(b_hbm.at[svec.at[pl.ds(off, nrows)]],
                        bufb.at[pl.ds(0, nrows)])

        def row_body(r, carry):
            for l in range(BIG // 16):
                sl = pl.ds(l * 16, 16)
                va = bufa[r, sl]
                vb = bufb[r, sl]
                bufa[r, sl] = jnp.maximum(va + vb, 0.0)
            return carry

        lax.fori_loop(0, nrows, row_body, 0)
        pltpu.sync_copy(bufa.at[pl.ds(0, nrows)],
                        t_hbm.at[pl.ds(base + off, nrows)])

    def chunk_body(it, carry):
        do_chunk(it * _GCH, _GCH)
        return carry

    lax.fori_loop(0, _GFULL, chunk_body, 0)
    do_chunk(_GFULL * _GCH, _GTAIL)


def _gather_add_relu(a, b, dst, src):
    fn = functools.partial(
        pl.kernel,
        mesh=_mesh(),
        out_type=jax.ShapeDtypeStruct((E, BIG), F32),
        scratch_types=[
            pltpu.VMEM((_EPW,), jnp.int32),
            pltpu.VMEM((_EPW,), jnp.int32),
            pltpu.VMEM((_GCH, BIG), F32),
            pltpu.VMEM((_GCH, BIG), F32),
        ],
    )(_gather_body)
    return fn(a, b, dst, src)


_SROW = 80
_DROWS = E // _SROW
_ICH = 128
_NS0 = 624
_CW = 128


def _scatter_body(u_hbm, dst2_hbm, seg_hbm, acc, idxb, ubuf, zbuf):
    cid = lax.axis_index("c")
    sid = lax.axis_index("s")
    c0 = cid * _CW

    def zrow(r, carry):
        for l in range(_CW // 16):
            zbuf[r, pl.ds(l * 16, 16)] = jnp.zeros((16,), F32)
        return carry

    lax.fori_loop(0, 8, zrow, 0)

    nbase = sid * _NS0
    nrows_acc = jnp.where(sid == 15, N - 15 * _NS0, _NS0)

    def zcp(q, carry):
        pltpu.sync_copy(zbuf, acc.at[pl.ds(nbase + q * 8, 8)])
        return carry

    lax.fori_loop(0, nrows_acc // 8, zcp, 0)
    plsc.subcore_barrier()

    row0 = sid * _ICH
    nrows = jnp.where(sid == 15, _DROWS - 15 * _ICH, _ICH)

    @pl.when(sid < 15)
    def _():
        pltpu.sync_copy(dst2_hbm.at[pl.ds(row0, _ICH)], idxb)

    @pl.when(sid == 15)
    def _():
        tl = _DROWS - 15 * _ICH
        pltpu.sync_copy(dst2_hbm.at[pl.ds(15 * _ICH, tl)],
                        idxb.at[pl.ds(0, tl)])

    def acc_body(j, carry):
        erow = (row0 + j) * _SROW
        pltpu.sync_copy(u_hbm.at[pl.ds(erow, _SROW), pl.ds(c0, _CW)], ubuf)
        pltpu.sync_copy(ubuf, acc.at[idxb.at[j]], add=True)
        return carry

    lax.fori_loop(0, nrows, acc_body, 0)
    plsc.subcore_barrier()

    def wcp(q, carry):
        pltpu.sync_copy(acc.at[pl.ds(nbase + q * 8, 8)],
                        seg_hbm.at[pl.ds(nbase + q * 8, 8), pl.ds(c0, _CW)])
        return carry

    lax.fori_loop(0, nrows_acc // 8, wcp, 0)


def _scatter(u, dst2d):
    fn = functools.partial(
        pl.kernel,
        mesh=_mesh(),
        out_type=jax.ShapeDtypeStruct((N, BIG), F32),
        scratch_types=[
            pltpu.VMEM_SHARED((N, _CW), F32),
            pltpu.VMEM((_ICH, _SROW), jnp.int32),
            pltpu.VMEM((_SROW, _CW), F32),
            pltpu.VMEM((8, _CW), F32),
        ],
    )(_scatter_body)
    return fn(u, dst2d)


_CNT_ROWS0 = 64


def _cnt_body(dst2_hbm, out_hbm, acc_cnt, idxb, onesb, zbuf16):
    cid = lax.axis_index("c")
    sid = lax.axis_index("s")
    wid = sid * 2 + cid

    def zrow(r, carry):
        for l in range(_CW // 16):
            zbuf16[r, pl.ds(l * 16, 16)] = jnp.zeros((16,), F32)
        return carry

    lax.fori_loop(0, 8, zrow, 0)

    def onesrow(r, carry):
        for l in range(_CW // 16):
            onesb[r, pl.ds(l * 16, 16)] = jnp.ones((16,), F32)
        return carry

    lax.fori_loop(0, _SROW, onesrow, 0)

    zbase = sid * 632
    zn = jnp.where(sid == 15, N - 15 * 632, 632)

    def zcp(q, carry):
        pltpu.sync_copy(zbuf16, acc_cnt.at[pl.ds(zbase + q * 8, 8)])
        return carry

    lax.fori_loop(0, zn // 8, zcp, 0)
    plsc.subcore_barrier()

    wbase = wid * _CNT_ROWS0
    wn = jnp.where(wid == 31, _DROWS - 31 * _CNT_ROWS0, _CNT_ROWS0)

    @pl.when(wid < 31)
    def _():
        pltpu.sync_copy(dst2_hbm.at[pl.ds(wbase, _CNT_ROWS0)], idxb)

    @pl.when(wid == 31)
    def _():
        tl = _DROWS - 31 * _CNT_ROWS0
        pltpu.sync_copy(dst2_hbm.at[pl.ds(31 * _CNT_ROWS0, tl)],
                        idxb.at[pl.ds(0, tl)])

    def acc_body(j, carry):
        pltpu.sync_copy(onesb, acc_cnt.at[idxb.at[j]], add=True)
        return carry

    lax.fori_loop(0, wn, acc_body, 0)
    plsc.subcore_barrier()

    def wcp(q, carry):
        pltpu.sync_copy(acc_cnt.at[pl.ds(zbase + q * 8, 8)],
                        out_hbm.at[cid, pl.ds(zbase + q * 8, 8)])
        return carry

    lax.fori_loop(0, zn // 8, wcp, 0)


def _cnt_sc(dst2d):
    fn = functools.partial(
        pl.kernel,
        mesh=_mesh(),
        out_type=jax.ShapeDtypeStruct((2, N, _CW), F32),
        scratch_types=[
            pltpu.VMEM_SHARED((N, _CW), F32),
            pltpu.VMEM((_CNT_ROWS0, _SROW), jnp.int32),
            pltpu.VMEM((_SROW, _CW), F32),
            pltpu.VMEM((8, _CW), F32),
        ],
    )(_cnt_body)
    return fn(dst2d)


def kernel(x, edge_index, enc_w0, enc_b0, enc_w1, enc_b1,
           dec_w0, dec_b0, dec_w1, dec_b1, dec_w2, dec_b2,
           mu_w, mu_b, var_w, var_b, ad_w, ad_b,
           d1_w, d1_b, d2_w, d2_b, ab_w, ab_b):
    src = edge_index[0]
    dst = edge_index[1]
    dst2d = dst.reshape(_DROWS, _SROW)

    w0t = enc_w0.T
    b0r = enc_b0.reshape(1, BIG)
    w1t = enc_w1.T
    b1r = enc_b1.reshape(1, BIG)
    adwt = ad_w.reshape(H, H, K, BIG).transpose(2, 3, 0, 1).reshape(K, BIG, H * H)
    adbt = ad_b.reshape(H, H, K).transpose(2, 0, 1).reshape(K, 1, H * H)
    d1wt = d1_w.reshape(H, K, BIG).transpose(1, 2, 0)
    d1br = d1_b.reshape(H, K).T.reshape(K, 1, H)
    d2wt = d2_w.reshape(H, K, BIG).transpose(1, 2, 0)
    d2br = d2_b.reshape(H, K).T.reshape(K, 1, H)
    abwt = ab_w.reshape(H, K, BIG).transpose(1, 2, 0)
    abbr = ab_b.reshape(H, K).T.reshape(K, 1, H)
    muwt = mu_w.T
    mubr = mu_b.reshape(1, H)
    vwt = var_w.T
    vbr = var_b.reshape(1, H)
    d0t = dec_w0.T
    d0br = dec_b0.reshape(1, BIG)
    dw1t = dec_w1.T
    db1r = dec_b1.reshape(1, BIG)
    dw2t = dec_w2.T
    db2r = dec_b2.reshape(1, D_OUT)

    eps = jax.random.normal(jax.random.key(42), (N, H), dtype=F32)

    a1, b1v = _pre_enc(x, w0t, b0r)
    t1 = _gather_add_relu(a1, b1v, dst, src)
    u1 = _mm_relu(t1, w1t, b1r)
    cntp = _cnt_sc(dst2d)
    cnt0 = cntp[0]
    cnt1 = cntp[1]
    seg1 = _scatter(u1, dst2d)
    mu, lv, z0, z, ld128, a2, b2v = _post_enc(
        seg1, cnt0, cnt1, eps, adwt, adbt, d1wt, d1br, d2wt, d2br, abwt, abbr,
        muwt, mubr, vwt, vbr, d0t, d0br)
    t2 = _gather_add_relu(a2, b2v, dst, src)
    u2 = _mm_relu(t2, dw1t, db1r)
    seg2 = _scatter(u2, dst2d)
    x_dec = _dec_out(seg2, cnt0, cnt1, dw2t, db2r)

    return x_dec, mu, lv, ld128[:, 0], z0, z
